# Initial kernel scaffold; baseline (speedup 1.0000x reference)
import jax
import jax.numpy as jnp
from jax.experimental import pallas as pl


def kernel(x, edge_index, batch, conv1_Wrel, conv1_brel, conv1_Wroot, p1, conv2_Wrel, conv2_brel, conv2_Wroot, p2, conv3_Wrel, conv3_brel, conv3_Wroot, p3, lin1_W, lin1_b, lin2_W, lin2_b, lin3_W, lin3_b):
    raise NotImplementedError("write your pallas kernel here")


# speedup vs baseline: 48.8596x; 48.8596x over previous
import functools

import jax
import jax.numpy as jnp
from jax import lax
from jax.experimental import pallas as pl
from jax.experimental.pallas import tpu as pltpu
from jax.experimental.pallas import tpu_sc as plsc

_B = 100
_NPG = 100
_D = 128
_E = 320000
_K = (50, 25, 13)
_PADR = 104
_AROWS = _B * _PADR
_AFLAT = _AROWS * _NPG
_SH = 1040384
_NW = 32
_EW = _E // _NW
_NR = 79
_ZCH = _SH // 16


def _adj_body(src_hbm, dst_hbm, zero_hbm, out_hbm,
              src_v, dst_v, ones_v, acc_sh, sem):
    c = lax.axis_index("c")
    s = lax.axis_index("s")

    pltpu.sync_copy(zero_hbm.at[pl.ds(s * _ZCH, _ZCH)],
                    acc_sh.at[pl.ds(s * _ZCH, _ZCH)])

    base = (c * 16 + s) * _EW
    pltpu.sync_copy(src_hbm.at[pl.ds(base, _EW)], src_v)
    pltpu.sync_copy(dst_hbm.at[pl.ds(base, _EW)], dst_v)

    ones_v[...] = jnp.ones((16,), jnp.float32)

    plsc.subcore_barrier()

    def step(t, carry):
        sv = src_v[pl.ds(t * 16, 16)]
        dv = dst_v[pl.ds(t * 16, 16)]
        gq = lax.shift_right_logical(dv * 5243, 19)
        sq = lax.shift_right_logical(sv * 5243, 19)
        row = dv + gq * (_PADR - _NPG)
        col = sv - sq * _NPG
        f = row * _NPG + col
        pltpu.sync_copy(ones_v, acc_sh.at[f], add=True)
        return carry

    lax.fori_loop(0, _EW // 16, step, 0)

    plsc.subcore_barrier()

    @pl.when(s < 13)
    def _():
        pltpu.sync_copy(acc_sh.at[pl.ds(s * 80000, 80000)],
                        out_hbm.at[pl.ds(c * _AFLAT + s * 80000, 80000)])


@functools.cache
def _get_adj_kernel():
    return pl.kernel(
        _adj_body,
        mesh=plsc.VectorSubcoreMesh(core_axis_name="c", subcore_axis_name="s"),
        out_type=jax.ShapeDtypeStruct((2 * _AFLAT,), jnp.float32),
        scratch_types=[
            pltpu.VMEM((_EW,), jnp.int32),
            pltpu.VMEM((_EW,), jnp.int32),
            pltpu.VMEM((16,), jnp.float32),
            pltpu.VMEM_SHARED((_SH,), jnp.float32),
            pltpu.SemaphoreType.DMA,
        ],
    )


def _mm(a, b):
    return lax.dot_general(a, b, (((1,), (0,)), ((), ())),
                           preferred_element_type=jnp.float32)


def _fwd_body(at3_ref, xp_ref,
              w1r_ref, b1_ref, w1o_ref, p1_ref,
              w2r_ref, b2_ref, w2o_ref, p2_ref,
              w3r_ref, b3_ref, w3o_ref, p3_ref,
              l1w_ref, l1b_ref, l2w_ref, l2b_ref, l3w_ref, l3b_ref,
              out_ref,
              asum_ref, xm_ref, agg_ref, h_ref, nid_ref, roa_ref, rob_ref):
    asum_ref[...] = at3_ref[0] + at3_ref[1]

    stage_params = (
        (w1r_ref, b1_ref, w1o_ref, p1_ref),
        (w2r_ref, b2_ref, w2o_ref, p2_ref),
        (w3r_ref, b3_ref, w3o_ref, p3_ref),
    )

    for t in range(3):
        wr_ref, b_ref, wo_ref, p_ref = stage_params[t]
        k = _K[t]
        xsrc = xp_ref if t == 0 else xm_ref

        def a_body(g, carry):
            b = g * _PADR
            atg = asum_ref[pl.ds(b, _NPG), :]
            xg = xsrc[pl.ds(b, _NPG), :]
            agg_ref[pl.ds(b, _NPG), :] = _mm(atg, xg)
            return carry

        lax.fori_loop(0, _B, a_body, 0)

        bias = b_ref[...][None, :]
        h_ref[...] = jnp.maximum(
            _mm(agg_ref[...], wr_ref[...]) + bias + _mm(xsrc[...], wo_ref[...]),
            0.0)

        p2d = p_ref[...][None, :]
        nrm = jnp.sqrt(jnp.sum(p2d * p2d)) + 1e-16

        def g_body(g, carry, t=t, k=k, p2d=p2d, nrm=nrm):
            b = g * _PADR
            hg = h_ref[pl.ds(b, _NPG), :]
            sdot = lax.dot_general(hg, p2d, (((1,), (1,)), ((), ())),
                                   preferred_element_type=jnp.float32)
            scol_t = jnp.tanh(sdot / nrm)
            if t == 0:
                ncol = lax.broadcasted_iota(
                    jnp.int32, (_NPG, 1), 0).astype(jnp.float32)
                scol_m = scol_t
            else:
                nrow_prev = nid_ref[pl.ds(g, 1), :]
                ncol = lax.transpose(nrow_prev, (1, 0))
                scol_m = jnp.where(ncol < 1e8, scol_t, -2.0)
            srow_m = lax.transpose(scol_m, (1, 0))
            nrow = lax.transpose(ncol, (1, 0))
            beat = jnp.where(
                (srow_m > scol_m) | ((srow_m == scol_m) & (nrow < ncol)),
                1.0, 0.0)
            rank_col = jnp.sum(beat, axis=1, keepdims=True)
            keep_col = jnp.where(rank_col < k, 1.0, 0.0)
            rank_row = 99.0 - jnp.sum(beat, axis=0, keepdims=True)
            nid_row = jnp.where(rank_row < k, rank_row, 1e9)
            nid_ref[pl.ds(g, 1), :] = nid_row
            xn = hg * scol_t * keep_col
            xm_ref[pl.ds(b, _NPG), :] = xn
            rmax = jnp.max(jnp.where(keep_col > 0, xn, -1e30),
                           axis=0, keepdims=True)
            rmean = jnp.sum(xn, axis=0, keepdims=True) / k
            if t == 0:
                roa_ref[pl.ds(g, 1), :] = rmax
                rob_ref[pl.ds(g, 1), :] = rmean
            else:
                roa_ref[pl.ds(g, 1), :] = roa_ref[pl.ds(g, 1), :] + rmax
                rob_ref[pl.ds(g, 1), :] = rob_ref[pl.ds(g, 1), :] + rmean
            return carry

        lax.fori_loop(0, _B, g_body, 0)

    ro = jnp.concatenate([roa_ref[...], rob_ref[...]], axis=1)
    z1 = jnp.maximum(_mm(ro, l1w_ref[...]) + l1b_ref[...][None, :], 0.0)
    z2 = jnp.maximum(_mm(z1, l2w_ref[...]) + l2b_ref[...][None, :], 0.0)
    z3 = _mm(z2, l3w_ref[...]) + l3b_ref[...][None, :]
    zmax = jnp.max(z3, axis=1, keepdims=True)
    zs = z3 - zmax
    lse = jnp.log(jnp.sum(jnp.exp(zs), axis=1, keepdims=True))
    out_ref[...] = zs - lse


def kernel(x, edge_index, batch, conv1_Wrel, conv1_brel, conv1_Wroot, p1,
           conv2_Wrel, conv2_brel, conv2_Wroot, p2,
           conv3_Wrel, conv3_brel, conv3_Wroot, p3,
           lin1_W, lin1_b, lin2_W, lin2_b, lin3_W, lin3_b):
    src = edge_index[0]
    dst = edge_index[1]
    zeros_init = jnp.zeros((_SH,), jnp.float32)

    ap = _get_adj_kernel()(src, dst, zeros_init)
    at3 = ap.reshape(2, _AROWS, _NPG)

    xp = jnp.pad(x.reshape(_B, _NPG, _D), ((0, 0), (0, _PADR - _NPG), (0, 0)))
    xp = xp.reshape(_AROWS, _D)

    out = pl.pallas_call(
        _fwd_body,
        out_shape=jax.ShapeDtypeStruct((_B, 10), jnp.float32),
        scratch_shapes=[
            pltpu.VMEM((_AROWS, _NPG), jnp.float32),
            pltpu.VMEM((_AROWS, _D), jnp.float32),
            pltpu.VMEM((_AROWS, _D), jnp.float32),
            pltpu.VMEM((_AROWS, _D), jnp.float32),
            pltpu.VMEM((_B, _NPG), jnp.float32),
            pltpu.VMEM((_B, _D), jnp.float32),
            pltpu.VMEM((_B, _D), jnp.float32),
        ],
    )(at3, xp, conv1_Wrel, conv1_brel, conv1_Wroot, p1,
      conv2_Wrel, conv2_brel, conv2_Wroot, p2,
      conv3_Wrel, conv3_brel, conv3_Wroot, p3,
      lin1_W, lin1_b, lin2_W, lin2_b, lin3_W, lin3_b)
    return out


# speedup vs baseline: 48.9154x; 1.0011x over previous
import functools

import jax
import jax.numpy as jnp
from jax import lax
from jax.experimental import pallas as pl
from jax.experimental.pallas import tpu as pltpu
from jax.experimental.pallas import tpu_sc as plsc

_B = 100
_NPG = 100
_D = 128
_E = 320000
_K = (50, 25, 13)
_PADR = 104
_AROWS = _B * _PADR
_AFLAT = _AROWS * _NPG
_SH = 1040384
_NW = 32
_EW = _E // _NW
_NR = 79
_ZCH = _SH // 16


def _adj_body(src_hbm, dst_hbm, zero_hbm, out_hbm,
              src_v, dst_v, ones_v, acc_sh, sem):
    c = lax.axis_index("c")
    s = lax.axis_index("s")

    pltpu.sync_copy(zero_hbm.at[pl.ds(s * _ZCH, _ZCH)],
                    acc_sh.at[pl.ds(s * _ZCH, _ZCH)])

    base = (c * 16 + s) * _EW
    pltpu.sync_copy(src_hbm.at[pl.ds(base, _EW)], src_v)
    pltpu.sync_copy(dst_hbm.at[pl.ds(base, _EW)], dst_v)

    ones_v[...] = jnp.ones((16,), jnp.float32)

    plsc.subcore_barrier()

    def step(t, carry):
        sv = src_v[pl.ds(t * 16, 16)]
        dv = dst_v[pl.ds(t * 16, 16)]
        gq = lax.shift_right_logical(dv * 5243, 19)
        sq = lax.shift_right_logical(sv * 5243, 19)
        row = dv + gq * (_PADR - _NPG)
        col = sv - sq * _NPG
        f = row * _NPG + col
        pltpu.sync_copy(ones_v, acc_sh.at[f], add=True)
        return carry

    lax.fori_loop(0, _EW // 16, step, 0)

    plsc.subcore_barrier()

    @pl.when(s < 13)
    def _():
        pltpu.sync_copy(acc_sh.at[pl.ds(s * 80000, 80000)],
                        out_hbm.at[pl.ds(c * _AFLAT + s * 80000, 80000)])


@functools.cache
def _get_adj_kernel():
    return pl.kernel(
        _adj_body,
        mesh=plsc.VectorSubcoreMesh(core_axis_name="c", subcore_axis_name="s"),
        out_type=jax.ShapeDtypeStruct((2 * _AFLAT,), jnp.float32),
        scratch_types=[
            pltpu.VMEM((_EW,), jnp.int32),
            pltpu.VMEM((_EW,), jnp.int32),
            pltpu.VMEM((16,), jnp.float32),
            pltpu.VMEM_SHARED((_SH,), jnp.float32),
            pltpu.SemaphoreType.DMA,
        ],
    )


def _mm(a, b):
    return lax.dot_general(a, b, (((1,), (0,)), ((), ())),
                           preferred_element_type=jnp.float32)


def _fwd_body(at3_ref, xp_ref,
              w1r_ref, b1_ref, w1o_ref, p1_ref,
              w2r_ref, b2_ref, w2o_ref, p2_ref,
              w3r_ref, b3_ref, w3o_ref, p3_ref,
              l1w_ref, l1b_ref, l2w_ref, l2b_ref, l3w_ref, l3b_ref,
              out_ref,
              asum_ref, xm_ref, agg_ref, h_ref, nid_ref, roa_ref, rob_ref):
    asum_ref[...] = at3_ref[0] + at3_ref[1]

    stage_params = (
        (w1r_ref, b1_ref, w1o_ref, p1_ref),
        (w2r_ref, b2_ref, w2o_ref, p2_ref),
        (w3r_ref, b3_ref, w3o_ref, p3_ref),
    )

    for t in range(3):
        wr_ref, b_ref, wo_ref, p_ref = stage_params[t]
        k = _K[t]
        xsrc = xp_ref if t == 0 else xm_ref

        def a_body(g, carry):
            b = g * _PADR
            atg = asum_ref[pl.ds(b, _NPG), :]
            xg = xsrc[pl.ds(b, _NPG), :]
            agg_ref[pl.ds(b, _NPG), :] = _mm(atg, xg)
            return carry

        lax.fori_loop(0, _B, a_body, 0)

        bias = b_ref[...][None, :]
        h_ref[...] = jnp.maximum(
            _mm(agg_ref[...], wr_ref[...]) + (bias + _mm(xsrc[...], wo_ref[...])),
            0.0)

        p2d = p_ref[...][None, :]
        nrm = jnp.sqrt(jnp.sum(p2d * p2d)) + 1e-16

        def g_body(g, carry, t=t, k=k, p2d=p2d, nrm=nrm):
            b = g * _PADR
            hg = h_ref[pl.ds(b, _NPG), :]
            sdot = lax.dot_general(hg, p2d, (((1,), (1,)), ((), ())),
                                   preferred_element_type=jnp.float32)
            scol_t = jnp.tanh(sdot / nrm)
            if t == 0:
                ncol = lax.broadcasted_iota(
                    jnp.int32, (_NPG, 1), 0).astype(jnp.float32)
                scol_m = scol_t
            else:
                nrow_prev = nid_ref[pl.ds(g, 1), :]
                ncol = lax.transpose(nrow_prev, (1, 0))
                scol_m = jnp.where(ncol < 1e8, scol_t, -2.0)
            srow_m = lax.transpose(scol_m, (1, 0))
            nrow = lax.transpose(ncol, (1, 0))
            beat = jnp.where(
                (srow_m > scol_m) | ((srow_m == scol_m) & (nrow < ncol)),
                1.0, 0.0)
            rank_col = jnp.sum(beat, axis=1, keepdims=True)
            keep_col = jnp.where(rank_col < k, 1.0, 0.0)
            rank_row = 99.0 - jnp.sum(beat, axis=0, keepdims=True)
            nid_row = jnp.where(rank_row < k, rank_row, 1e9)
            nid_ref[pl.ds(g, 1), :] = nid_row
            xn = hg * scol_t * keep_col
            xm_ref[pl.ds(b, _NPG), :] = xn
            rmax = jnp.max(jnp.where(keep_col > 0, xn, -1e30),
                           axis=0, keepdims=True)
            rmean = jnp.sum(xn, axis=0, keepdims=True) / k
            if t == 0:
                roa_ref[pl.ds(g, 1), :] = rmax
                rob_ref[pl.ds(g, 1), :] = rmean
            else:
                roa_ref[pl.ds(g, 1), :] = roa_ref[pl.ds(g, 1), :] + rmax
                rob_ref[pl.ds(g, 1), :] = rob_ref[pl.ds(g, 1), :] + rmean
            return carry

        lax.fori_loop(0, _B, g_body, 0)

    ro = jnp.concatenate([roa_ref[...], rob_ref[...]], axis=1)
    z1 = jnp.maximum(_mm(ro, l1w_ref[...]) + l1b_ref[...][None, :], 0.0)
    z2 = jnp.maximum(_mm(z1, l2w_ref[...]) + l2b_ref[...][None, :], 0.0)
    z3 = _mm(z2, l3w_ref[...]) + l3b_ref[...][None, :]
    zmax = jnp.max(z3, axis=1, keepdims=True)
    zs = z3 - zmax
    lse = jnp.log(jnp.sum(jnp.exp(zs), axis=1, keepdims=True))
    out_ref[...] = zs - lse


def kernel(x, edge_index, batch, conv1_Wrel, conv1_brel, conv1_Wroot, p1,
           conv2_Wrel, conv2_brel, conv2_Wroot, p2,
           conv3_Wrel, conv3_brel, conv3_Wroot, p3,
           lin1_W, lin1_b, lin2_W, lin2_b, lin3_W, lin3_b):
    src = edge_index[0]
    dst = edge_index[1]
    zeros_init = jnp.zeros((_SH,), jnp.float32)

    ap = _get_adj_kernel()(src, dst, zeros_init)
    at3 = ap.reshape(2, _AROWS, _NPG)

    xp = jnp.pad(x.reshape(_B, _NPG, _D), ((0, 0), (0, _PADR - _NPG), (0, 0)))
    xp = xp.reshape(_AROWS, _D)

    out = pl.pallas_call(
        _fwd_body,
        out_shape=jax.ShapeDtypeStruct((_B, 10), jnp.float32),
        scratch_shapes=[
            pltpu.VMEM((_AROWS, _NPG), jnp.float32),
            pltpu.VMEM((_AROWS, _D), jnp.float32),
            pltpu.VMEM((_AROWS, _D), jnp.float32),
            pltpu.VMEM((_AROWS, _D), jnp.float32),
            pltpu.VMEM((_B, _NPG), jnp.float32),
            pltpu.VMEM((_B, _D), jnp.float32),
            pltpu.VMEM((_B, _D), jnp.float32),
        ],
    )(at3, xp, conv1_Wrel, conv1_brel, conv1_Wroot, p1,
      conv2_Wrel, conv2_brel, conv2_Wroot, p2,
      conv3_Wrel, conv3_brel, conv3_Wroot, p3,
      lin1_W, lin1_b, lin2_W, lin2_b, lin3_W, lin3_b)
    return out
